# Initial kernel scaffold; baseline (speedup 1.0000x reference)
import jax
import jax.numpy as jnp
from jax.experimental import pallas as pl


def kernel(x, edge_index, batch, W1a, b1a, W2a, b2a, W1b, b1b, W2b, b2b, Wl, bl):
    raise NotImplementedError("write your pallas kernel here")


# speedup vs baseline: 6.1033x; 6.1033x over previous
import functools

import jax
import jax.numpy as jnp
from jax import lax
from jax.experimental import pallas as pl
from jax.experimental.pallas import tpu as pltpu
from jax.experimental.pallas import tpu_sc as plsc

N = 10000
E = 320000
D = 128
H = 16
G = 64

NC = 2
NS = 16
NW = NC * NS
EPW = E // NW
K = 80
NCHUNK = EPW // K
NPAD = 10240
NPS = NPAD // NS

BLK = 2000
NBLK = N // BLK


_SC_MESH = plsc.VectorSubcoreMesh(core_axis_name="c", subcore_axis_name="s")


@functools.partial(
    pl.kernel,
    out_type=jax.ShapeDtypeStruct((NC, NPAD, H), jnp.float32),
    mesh=_SC_MESH,
    scratch_types=[
        pltpu.VMEM((K,), jnp.int32),
        pltpu.VMEM((K,), jnp.int32),
        pltpu.VMEM((K, H), jnp.float32),
        pltpu.VMEM((NPS, H), jnp.float32),
        pltpu.VMEM_SHARED((NPAD, H), jnp.float32),
        pltpu.SemaphoreType.DMA,
    ],
    compiler_params=pltpu.CompilerParams(use_tc_tiling_on_sc=False),
)
def _edge_agg(src_hbm, dst_hbm, y_hbm, out_hbm,
              src_v, dst_v, rows_v, zbuf, acc_sh, sem):
    c = lax.axis_index("c")
    s = lax.axis_index("s")

    def _zero(i, carry):
        zbuf[i, :] = jnp.zeros((H,), jnp.float32)
        return carry

    lax.fori_loop(0, NPS, _zero, 0)
    pltpu.sync_copy(zbuf, acc_sh.at[pl.ds(s * NPS, NPS)])
    plsc.subcore_barrier()

    wid = c * NS + s

    def _chunk(j, carry):
        base = wid * EPW + j * K
        pltpu.sync_copy(src_hbm.at[pl.ds(base, K)], src_v)
        pltpu.sync_copy(dst_hbm.at[pl.ds(base, K)], dst_v)
        pltpu.async_copy(y_hbm.at[src_v], rows_v, sem).wait()
        pltpu.sync_copy(rows_v, acc_sh.at[dst_v], add=True)
        return carry

    lax.fori_loop(0, NCHUNK, _chunk, 0)
    plsc.subcore_barrier()

    pltpu.sync_copy(acc_sh.at[pl.ds(s * NPS, NPS)],
                    out_hbm.at[c, pl.ds(s * NPS, NPS)])


def _proj_body(x_ref, w_ref, o_ref):
    o_ref[...] = jnp.dot(x_ref[...], w_ref[...],
                         preferred_element_type=jnp.float32)


_proj = pl.pallas_call(
    _proj_body,
    grid=(NBLK,),
    in_specs=[
        pl.BlockSpec((BLK, D), lambda i: (i, 0)),
        pl.BlockSpec((D, H), lambda i: (0, 0)),
    ],
    out_specs=pl.BlockSpec((BLK, H), lambda i: (i, 0)),
    out_shape=jax.ShapeDtypeStruct((N, H), jnp.float32),
)


def _mlp1_body(y_ref, p0_ref, p1_ref, b1_ref, w2_ref, b2_ref, o_ref):
    z = jnp.maximum(y_ref[...] + p0_ref[...] + p1_ref[...] + b1_ref[...], 0.0)
    t = jnp.dot(z, w2_ref[...], preferred_element_type=jnp.float32) + b2_ref[...]
    o_ref[...] = jnp.maximum(t, 0.0)


_mlp1 = pl.pallas_call(
    _mlp1_body,
    grid=(NBLK,),
    in_specs=[
        pl.BlockSpec((BLK, H), lambda i: (i, 0)),
        pl.BlockSpec((BLK, H), lambda i: (i, 0)),
        pl.BlockSpec((BLK, H), lambda i: (i, 0)),
        pl.BlockSpec((1, H), lambda i: (0, 0)),
        pl.BlockSpec((H, H), lambda i: (0, 0)),
        pl.BlockSpec((1, H), lambda i: (0, 0)),
    ],
    out_specs=pl.BlockSpec((BLK, H), lambda i: (i, 0)),
    out_shape=jax.ShapeDtypeStruct((N, H), jnp.float32),
)


def _mlp2_body(h_ref, q0_ref, q1_ref, b_ref, w1_ref, b1_ref, w2_ref, b2_ref,
               wl_ref, bl_ref, o_ref, sums, cnt):
    i = pl.program_id(0)

    @pl.when(i == 0)
    def _():
        sums[...] = jnp.zeros_like(sums)
        cnt[...] = jnp.zeros_like(cnt)

    z = h_ref[...] + q0_ref[...] + q1_ref[...]
    t = jnp.maximum(
        jnp.dot(z, w1_ref[...], preferred_element_type=jnp.float32)
        + b1_ref[...], 0.0)
    u = jnp.dot(t, w2_ref[...], preferred_element_type=jnp.float32) + b2_ref[...]

    gids = lax.broadcasted_iota(jnp.int32, (G, BLK), 0)
    onehot_t = (b_ref[0] == gids).astype(jnp.float32)
    sums[...] += jnp.dot(onehot_t, u, preferred_element_type=jnp.float32)
    cnt[...] += jnp.dot(onehot_t, jnp.ones((BLK, 1), jnp.float32),
                        preferred_element_type=jnp.float32)

    @pl.when(i == pl.num_programs(0) - 1)
    def _():
        v = jnp.dot(sums[...], wl_ref[...], preferred_element_type=jnp.float32)
        o_ref[...] = v / jnp.maximum(cnt[...], 1.0) + bl_ref[...]


_mlp2pool = pl.pallas_call(
    _mlp2_body,
    grid=(NBLK,),
    in_specs=[
        pl.BlockSpec((BLK, H), lambda i: (i, 0)),
        pl.BlockSpec((BLK, H), lambda i: (i, 0)),
        pl.BlockSpec((BLK, H), lambda i: (i, 0)),
        pl.BlockSpec((1, 1, BLK), lambda i: (i, 0, 0)),
        pl.BlockSpec((H, H), lambda i: (0, 0)),
        pl.BlockSpec((1, H), lambda i: (0, 0)),
        pl.BlockSpec((H, H), lambda i: (0, 0)),
        pl.BlockSpec((1, H), lambda i: (0, 0)),
        pl.BlockSpec((H, 1), lambda i: (0, 0)),
        pl.BlockSpec((1, 1), lambda i: (0, 0)),
    ],
    out_specs=pl.BlockSpec((G, 1), lambda i: (0, 0)),
    out_shape=jax.ShapeDtypeStruct((G, 1), jnp.float32),
    scratch_shapes=[
        pltpu.VMEM((G, H), jnp.float32),
        pltpu.VMEM((G, 1), jnp.float32),
    ],
)


def kernel(x, edge_index, batch, W1a, b1a, W2a, b2a, W1b, b1b, W2b, b2b, Wl, bl):
    src = edge_index[0].astype(jnp.int32)
    dst = edge_index[1].astype(jnp.int32)
    batch3 = batch.astype(jnp.int32).reshape(NBLK, 1, BLK)
    b1a2 = b1a.reshape(1, H)
    b2a2 = b2a.reshape(1, H)
    b1b2 = b1b.reshape(1, H)
    b2b2 = b2b.reshape(1, H)
    bl2 = bl.reshape(1, 1)

    y = _proj(x, W1a)
    p = _edge_agg(src, dst, y)
    h = _mlp1(y, p[0, :N], p[1, :N], b1a2, W2a, b2a2)
    q = _edge_agg(src, dst, h)
    out = _mlp2pool(h, q[0, :N], q[1, :N], batch3,
                    W1b, b1b2, W2b, b2b2, Wl, bl2)
    return out


# speedup vs baseline: 12.4781x; 2.0445x over previous
import functools

import jax
import jax.numpy as jnp
from jax import lax
from jax.experimental import pallas as pl
from jax.experimental.pallas import tpu as pltpu
from jax.experimental.pallas import tpu_sc as plsc

N = 10000
E = 320000
D = 128
H = 16
G = 64

NC = 2
NS = 16
NW = NC * NS
EPW = E // NW
K = 128
NCHUNK = 80
EPAD = NW * NCHUNK * K
NBUF = 4
NPAD = 10240
NPS = NPAD // NS

BLK = 2000
NBLK = N // BLK


_SC_MESH = plsc.VectorSubcoreMesh(core_axis_name="c", subcore_axis_name="s")


@functools.partial(
    pl.kernel,
    out_type=jax.ShapeDtypeStruct((NC, NPAD, H), jnp.float32),
    mesh=_SC_MESH,
    scratch_types=[
        pltpu.VMEM((NCHUNK, K), jnp.int32),
        pltpu.VMEM((NCHUNK, K), jnp.int32),
        [pltpu.VMEM((K, H), jnp.float32) for _ in range(NBUF)],
        pltpu.VMEM((NPS, H), jnp.float32),
        pltpu.VMEM_SHARED((NPAD, H), jnp.float32),
        [pltpu.SemaphoreType.DMA for _ in range(NBUF)],
        [pltpu.SemaphoreType.DMA for _ in range(NBUF)],
    ],
    compiler_params=pltpu.CompilerParams(use_tc_tiling_on_sc=False),
)
def _edge_agg(src_hbm, dst_hbm, y_hbm, out_hbm,
              src_all, dst_all, rows, zbuf, acc_sh, gsems, ssems):
    c = lax.axis_index("c")
    s = lax.axis_index("s")
    wid = c * NS + s

    pltpu.sync_copy(src_hbm.at[wid], src_all)
    pltpu.sync_copy(dst_hbm.at[wid], dst_all)

    def _zero(i, carry):
        zbuf[i, :] = jnp.zeros((H,), jnp.float32)
        return carry

    lax.fori_loop(0, NPS, _zero, 0)
    pltpu.sync_copy(zbuf, acc_sh.at[pl.ds(s * NPS, NPS)])
    plsc.subcore_barrier()

    def _iter(i, carry):
        j0 = i * NBUF
        gds = []
        for b in range(NBUF):
            @pl.when(i > 0)
            def _(b=b, j0=j0):
                pltpu.make_async_copy(
                    rows[b], acc_sh.at[dst_all.at[j0 - NBUF + b]], ssems[b]
                ).wait()

            gds.append(
                pltpu.async_copy(y_hbm.at[src_all.at[j0 + b]], rows[b], gsems[b]))
        for b in range(NBUF):
            gds[b].wait()
            pltpu.async_copy(rows[b], acc_sh.at[dst_all.at[j0 + b]], ssems[b],
                             add=True)
        return carry

    lax.fori_loop(0, NCHUNK // NBUF, _iter, 0)
    for b in range(NBUF):
        pltpu.make_async_copy(
            rows[b], acc_sh.at[dst_all.at[NCHUNK - NBUF + b]], ssems[b]).wait()

    plsc.subcore_barrier()
    pltpu.sync_copy(acc_sh.at[pl.ds(s * NPS, NPS)],
                    out_hbm.at[c, pl.ds(s * NPS, NPS)])


def _proj_body(x_ref, w_ref, o_ref):
    o_ref[...] = jnp.dot(x_ref[...], w_ref[...],
                         preferred_element_type=jnp.float32,
                         precision=lax.Precision.HIGHEST)


_proj = pl.pallas_call(
    _proj_body,
    grid=(NBLK,),
    in_specs=[
        pl.BlockSpec((BLK, D), lambda i: (i, 0)),
        pl.BlockSpec((D, H), lambda i: (0, 0)),
    ],
    out_specs=pl.BlockSpec((BLK, H), lambda i: (i, 0)),
    out_shape=jax.ShapeDtypeStruct((N, H), jnp.float32),
)


def _mlp1_body(y_ref, p0_ref, p1_ref, b1_ref, w2_ref, b2_ref, o_ref):
    z = jnp.maximum(y_ref[...] + p0_ref[...] + p1_ref[...] + b1_ref[...], 0.0)
    t = jnp.dot(z, w2_ref[...], preferred_element_type=jnp.float32,
                         precision=lax.Precision.HIGHEST) + b2_ref[...]
    o_ref[...] = jnp.maximum(t, 0.0)


_mlp1 = pl.pallas_call(
    _mlp1_body,
    grid=(NBLK,),
    in_specs=[
        pl.BlockSpec((BLK, H), lambda i: (i, 0)),
        pl.BlockSpec((BLK, H), lambda i: (i, 0)),
        pl.BlockSpec((BLK, H), lambda i: (i, 0)),
        pl.BlockSpec((1, H), lambda i: (0, 0)),
        pl.BlockSpec((H, H), lambda i: (0, 0)),
        pl.BlockSpec((1, H), lambda i: (0, 0)),
    ],
    out_specs=pl.BlockSpec((BLK, H), lambda i: (i, 0)),
    out_shape=jax.ShapeDtypeStruct((N, H), jnp.float32),
)


def _mlp2_body(h_ref, q0_ref, q1_ref, b_ref, w1_ref, b1_ref, w2_ref, b2_ref,
               wl_ref, bl_ref, o_ref, sums, cnt):
    i = pl.program_id(0)

    @pl.when(i == 0)
    def _():
        sums[...] = jnp.zeros_like(sums)
        cnt[...] = jnp.zeros_like(cnt)

    z = h_ref[...] + q0_ref[...] + q1_ref[...]
    t = jnp.maximum(
        jnp.dot(z, w1_ref[...], preferred_element_type=jnp.float32,
                         precision=lax.Precision.HIGHEST)
        + b1_ref[...], 0.0)
    u = jnp.dot(t, w2_ref[...], preferred_element_type=jnp.float32,
                         precision=lax.Precision.HIGHEST) + b2_ref[...]

    gids = lax.broadcasted_iota(jnp.int32, (G, BLK), 0)
    onehot_t = (b_ref[0] == gids).astype(jnp.float32)
    sums[...] += jnp.dot(onehot_t, u, preferred_element_type=jnp.float32,
                         precision=lax.Precision.HIGHEST)
    cnt[...] += jnp.dot(onehot_t, jnp.ones((BLK, 1), jnp.float32),
                        preferred_element_type=jnp.float32,
                         precision=lax.Precision.HIGHEST)

    @pl.when(i == pl.num_programs(0) - 1)
    def _():
        v = jnp.dot(sums[...], wl_ref[...], preferred_element_type=jnp.float32,
                         precision=lax.Precision.HIGHEST)
        o_ref[...] = v / jnp.maximum(cnt[...], 1.0) + bl_ref[...]


_mlp2pool = pl.pallas_call(
    _mlp2_body,
    grid=(NBLK,),
    in_specs=[
        pl.BlockSpec((BLK, H), lambda i: (i, 0)),
        pl.BlockSpec((BLK, H), lambda i: (i, 0)),
        pl.BlockSpec((BLK, H), lambda i: (i, 0)),
        pl.BlockSpec((1, 1, BLK), lambda i: (i, 0, 0)),
        pl.BlockSpec((H, H), lambda i: (0, 0)),
        pl.BlockSpec((1, H), lambda i: (0, 0)),
        pl.BlockSpec((H, H), lambda i: (0, 0)),
        pl.BlockSpec((1, H), lambda i: (0, 0)),
        pl.BlockSpec((H, 1), lambda i: (0, 0)),
        pl.BlockSpec((1, 1), lambda i: (0, 0)),
    ],
    out_specs=pl.BlockSpec((G, 1), lambda i: (0, 0)),
    out_shape=jax.ShapeDtypeStruct((G, 1), jnp.float32),
    scratch_shapes=[
        pltpu.VMEM((G, H), jnp.float32),
        pltpu.VMEM((G, 1), jnp.float32),
    ],
)


def kernel(x, edge_index, batch, W1a, b1a, W2a, b2a, W1b, b1b, W2b, b2b, Wl, bl):
    src = edge_index[0].astype(jnp.int32)
    dst = edge_index[1].astype(jnp.int32)
    src3 = jnp.concatenate(
        [src, jnp.zeros((EPAD - E,), jnp.int32)]).reshape(NW, NCHUNK, K)
    dst3 = jnp.concatenate(
        [dst, jnp.full((EPAD - E,), N, jnp.int32)]).reshape(NW, NCHUNK, K)
    batch3 = batch.astype(jnp.int32).reshape(NBLK, 1, BLK)
    b1a2 = b1a.reshape(1, H)
    b2a2 = b2a.reshape(1, H)
    b1b2 = b1b.reshape(1, H)
    b2b2 = b2b.reshape(1, H)
    bl2 = bl.reshape(1, 1)

    y = _proj(x, W1a)
    p = _edge_agg(src3, dst3, y)
    h = _mlp1(y, p[0, :N], p[1, :N], b1a2, W2a, b2a2)
    q = _edge_agg(src3, dst3, h)
    out = _mlp2pool(h, q[0, :N], q[1, :N], batch3,
                    W1b, b1b2, W2b, b2b2, Wl, bl2)
    return out


# speedup vs baseline: 13.3032x; 1.0661x over previous
import functools

import jax
import jax.numpy as jnp
from jax import lax
from jax.experimental import pallas as pl
from jax.experimental.pallas import tpu as pltpu
from jax.experimental.pallas import tpu_sc as plsc

N = 10000
E = 320000
D = 128
H = 16
G = 64

NC = 2
NS = 16
NW = NC * NS
EPW = E // NW
K = 128
NCHUNK = 80
EPAD = NW * NCHUNK * K
NBUF = 8
NPAD = 10240
NPS = NPAD // NS

BLK = 2000
NBLK = N // BLK


_SC_MESH = plsc.VectorSubcoreMesh(core_axis_name="c", subcore_axis_name="s")


@functools.partial(
    pl.kernel,
    out_type=jax.ShapeDtypeStruct((NC, NPAD, H), jnp.float32),
    mesh=_SC_MESH,
    scratch_types=[
        pltpu.VMEM((NCHUNK, K), jnp.int32),
        pltpu.VMEM((NCHUNK, K), jnp.int32),
        [pltpu.VMEM((K, H), jnp.float32) for _ in range(NBUF)],
        pltpu.VMEM((NPS, H), jnp.float32),
        pltpu.VMEM_SHARED((NPAD, H), jnp.float32),
        [pltpu.SemaphoreType.DMA for _ in range(NBUF)],
        [pltpu.SemaphoreType.DMA for _ in range(NBUF)],
    ],
    compiler_params=pltpu.CompilerParams(use_tc_tiling_on_sc=False),
)
def _edge_agg(src_hbm, dst_hbm, y_hbm, out_hbm,
              src_all, dst_all, rows, zbuf, acc_sh, gsems, ssems):
    c = lax.axis_index("c")
    s = lax.axis_index("s")
    wid = c * NS + s

    pltpu.sync_copy(src_hbm.at[wid], src_all)
    pltpu.sync_copy(dst_hbm.at[wid], dst_all)

    def _zero(i, carry):
        zbuf[i, :] = jnp.zeros((H,), jnp.float32)
        return carry

    lax.fori_loop(0, NPS, _zero, 0)
    pltpu.sync_copy(zbuf, acc_sh.at[pl.ds(s * NPS, NPS)])
    plsc.subcore_barrier()

    def _iter(i, carry):
        j0 = i * NBUF
        gds = []
        for b in range(NBUF):
            @pl.when(i > 0)
            def _(b=b, j0=j0):
                pltpu.make_async_copy(
                    rows[b], acc_sh.at[dst_all.at[j0 - NBUF + b]], ssems[b]
                ).wait()

            gds.append(
                pltpu.async_copy(y_hbm.at[src_all.at[j0 + b]], rows[b], gsems[b]))
        for b in range(NBUF):
            gds[b].wait()
            pltpu.async_copy(rows[b], acc_sh.at[dst_all.at[j0 + b]], ssems[b],
                             add=True)
        return carry

    lax.fori_loop(0, NCHUNK // NBUF, _iter, 0)
    for b in range(NBUF):
        pltpu.make_async_copy(
            rows[b], acc_sh.at[dst_all.at[NCHUNK - NBUF + b]], ssems[b]).wait()

    plsc.subcore_barrier()
    pltpu.sync_copy(acc_sh.at[pl.ds(s * NPS, NPS)],
                    out_hbm.at[c, pl.ds(s * NPS, NPS)])


def _proj_body(x_ref, w_ref, o_ref):
    o_ref[...] = jnp.dot(x_ref[...], w_ref[...],
                         preferred_element_type=jnp.float32,
                         precision=lax.Precision.HIGHEST)


_proj = pl.pallas_call(
    _proj_body,
    grid=(NBLK,),
    in_specs=[
        pl.BlockSpec((BLK, D), lambda i: (i, 0)),
        pl.BlockSpec((D, H), lambda i: (0, 0)),
    ],
    out_specs=pl.BlockSpec((BLK, H), lambda i: (i, 0)),
    out_shape=jax.ShapeDtypeStruct((N, H), jnp.float32),
)


def _mlp1_body(y_ref, p0_ref, p1_ref, b1_ref, w2_ref, b2_ref, o_ref):
    z = jnp.maximum(y_ref[...] + p0_ref[0] + p1_ref[0] + b1_ref[...], 0.0)
    t = jnp.dot(z, w2_ref[...], preferred_element_type=jnp.float32,
                         precision=lax.Precision.HIGHEST) + b2_ref[...]
    o_ref[...] = jnp.maximum(t, 0.0)


_mlp1 = pl.pallas_call(
    _mlp1_body,
    grid=(NBLK,),
    in_specs=[
        pl.BlockSpec((BLK, H), lambda i: (i, 0)),
        pl.BlockSpec((1, BLK, H), lambda i: (0, i, 0)),
        pl.BlockSpec((1, BLK, H), lambda i: (1, i, 0)),
        pl.BlockSpec((1, H), lambda i: (0, 0)),
        pl.BlockSpec((H, H), lambda i: (0, 0)),
        pl.BlockSpec((1, H), lambda i: (0, 0)),
    ],
    out_specs=pl.BlockSpec((BLK, H), lambda i: (i, 0)),
    out_shape=jax.ShapeDtypeStruct((N, H), jnp.float32),
)


def _mlp2_body(h_ref, q0_ref, q1_ref, b_ref, w1_ref, b1_ref, w2_ref, b2_ref,
               wl_ref, bl_ref, o_ref, sums, cnt):
    i = pl.program_id(0)

    @pl.when(i == 0)
    def _():
        sums[...] = jnp.zeros_like(sums)
        cnt[...] = jnp.zeros_like(cnt)

    z = h_ref[...] + q0_ref[0] + q1_ref[0]
    t = jnp.maximum(
        jnp.dot(z, w1_ref[...], preferred_element_type=jnp.float32,
                         precision=lax.Precision.HIGHEST)
        + b1_ref[...], 0.0)
    u = jnp.dot(t, w2_ref[...], preferred_element_type=jnp.float32,
                         precision=lax.Precision.HIGHEST) + b2_ref[...]

    gids = lax.broadcasted_iota(jnp.int32, (G, BLK), 0)
    onehot_t = (b_ref[0] == gids).astype(jnp.float32)
    sums[...] += jnp.dot(onehot_t, u, preferred_element_type=jnp.float32,
                         precision=lax.Precision.HIGHEST)
    cnt[...] += jnp.dot(onehot_t, jnp.ones((BLK, 1), jnp.float32),
                        preferred_element_type=jnp.float32,
                         precision=lax.Precision.HIGHEST)

    @pl.when(i == pl.num_programs(0) - 1)
    def _():
        v = jnp.dot(sums[...], wl_ref[...], preferred_element_type=jnp.float32,
                         precision=lax.Precision.HIGHEST)
        o_ref[...] = v / jnp.maximum(cnt[...], 1.0) + bl_ref[...]


_mlp2pool = pl.pallas_call(
    _mlp2_body,
    grid=(NBLK,),
    in_specs=[
        pl.BlockSpec((BLK, H), lambda i: (i, 0)),
        pl.BlockSpec((1, BLK, H), lambda i: (0, i, 0)),
        pl.BlockSpec((1, BLK, H), lambda i: (1, i, 0)),
        pl.BlockSpec((1, 1, BLK), lambda i: (i, 0, 0)),
        pl.BlockSpec((H, H), lambda i: (0, 0)),
        pl.BlockSpec((1, H), lambda i: (0, 0)),
        pl.BlockSpec((H, H), lambda i: (0, 0)),
        pl.BlockSpec((1, H), lambda i: (0, 0)),
        pl.BlockSpec((H, 1), lambda i: (0, 0)),
        pl.BlockSpec((1, 1), lambda i: (0, 0)),
    ],
    out_specs=pl.BlockSpec((G, 1), lambda i: (0, 0)),
    out_shape=jax.ShapeDtypeStruct((G, 1), jnp.float32),
    scratch_shapes=[
        pltpu.VMEM((G, H), jnp.float32),
        pltpu.VMEM((G, 1), jnp.float32),
    ],
)


def kernel(x, edge_index, batch, W1a, b1a, W2a, b2a, W1b, b1b, W2b, b2b, Wl, bl):
    src = edge_index[0].astype(jnp.int32)
    dst = edge_index[1].astype(jnp.int32)
    src3 = jnp.concatenate(
        [src, jnp.zeros((EPAD - E,), jnp.int32)]).reshape(NW, NCHUNK, K)
    dst3 = jnp.concatenate(
        [dst, jnp.full((EPAD - E,), N, jnp.int32)]).reshape(NW, NCHUNK, K)
    batch3 = batch.astype(jnp.int32).reshape(NBLK, 1, BLK)
    b1a2 = b1a.reshape(1, H)
    b2a2 = b2a.reshape(1, H)
    b1b2 = b1b.reshape(1, H)
    b2b2 = b2b.reshape(1, H)
    bl2 = bl.reshape(1, 1)

    y = _proj(x, W1a)
    p = _edge_agg(src3, dst3, y)
    h = _mlp1(y, p, p, b1a2, W2a, b2a2)
    q = _edge_agg(src3, dst3, h)
    out = _mlp2pool(h, q, q, batch3,
                    W1b, b1b2, W2b, b2b2, Wl, bl2)
    return out


# speedup vs baseline: 19.3591x; 1.4552x over previous
import functools

import jax
import jax.numpy as jnp
from jax import lax
from jax.experimental import pallas as pl
from jax.experimental.pallas import tpu as pltpu
from jax.experimental.pallas import tpu_sc as plsc

N = 10000
E = 320000
D = 128
H = 16
G = 64

NC = 2
NS = 16
NW = NC * NS
EPW = E // NW
K = 128
NCHUNK = 80
EPAD = NW * NCHUNK * K
NBUF = 8
NPAD = 10240
NPS = NPAD // NS

BLK = 2000
NBLK = N // BLK


_SC_MESH = plsc.VectorSubcoreMesh(core_axis_name="c", subcore_axis_name="s")


@functools.partial(
    pl.kernel,
    out_type=jax.ShapeDtypeStruct((NC, NPAD, H), jnp.float32),
    mesh=_SC_MESH,
    scratch_types=[
        pltpu.VMEM((NCHUNK, K), jnp.int32),
        pltpu.VMEM((NCHUNK, K), jnp.int32),
        [pltpu.VMEM((K, H), jnp.float32) for _ in range(NBUF)],
        pltpu.VMEM((NPS, H), jnp.float32),
        pltpu.VMEM_SHARED((NPAD, H), jnp.float32),
        pltpu.VMEM_SHARED((N, H), jnp.float32),
        [pltpu.SemaphoreType.DMA for _ in range(NBUF)],
        [pltpu.SemaphoreType.DMA for _ in range(NBUF)],
    ],
    compiler_params=pltpu.CompilerParams(use_tc_tiling_on_sc=False),
)
def _edge_agg(src_hbm, dst_hbm, y_hbm, out_hbm,
              src_all, dst_all, rows, zbuf, acc_sh, y_sh, gsems, ssems):
    c = lax.axis_index("c")
    s = lax.axis_index("s")
    wid = c * NS + s

    pltpu.sync_copy(src_hbm.at[wid], src_all)
    pltpu.sync_copy(dst_hbm.at[wid], dst_all)
    YPS = N // NS
    pltpu.sync_copy(y_hbm.at[pl.ds(s * YPS, YPS)], y_sh.at[pl.ds(s * YPS, YPS)])

    def _zero(i, carry):
        zbuf[i, :] = jnp.zeros((H,), jnp.float32)
        return carry

    lax.fori_loop(0, NPS, _zero, 0)
    pltpu.sync_copy(zbuf, acc_sh.at[pl.ds(s * NPS, NPS)])
    plsc.subcore_barrier()

    def _iter(i, carry):
        j0 = i * NBUF
        gds = []
        for b in range(NBUF):
            @pl.when(i > 0)
            def _(b=b, j0=j0):
                pltpu.make_async_copy(
                    rows[b], acc_sh.at[dst_all.at[j0 - NBUF + b]], ssems[b]
                ).wait()

            gds.append(
                pltpu.async_copy(y_sh.at[src_all.at[j0 + b]], rows[b], gsems[b]))
        for b in range(NBUF):
            gds[b].wait()
            pltpu.async_copy(rows[b], acc_sh.at[dst_all.at[j0 + b]], ssems[b],
                             add=True)
        return carry

    lax.fori_loop(0, NCHUNK // NBUF, _iter, 0)
    for b in range(NBUF):
        pltpu.make_async_copy(
            rows[b], acc_sh.at[dst_all.at[NCHUNK - NBUF + b]], ssems[b]).wait()

    plsc.subcore_barrier()
    pltpu.sync_copy(acc_sh.at[pl.ds(s * NPS, NPS)],
                    out_hbm.at[c, pl.ds(s * NPS, NPS)])


def _proj_body(x_ref, w_ref, o_ref):
    o_ref[...] = jnp.dot(x_ref[...], w_ref[...],
                         preferred_element_type=jnp.float32,
                         precision=lax.Precision.HIGHEST)


_proj = pl.pallas_call(
    _proj_body,
    grid=(NBLK,),
    in_specs=[
        pl.BlockSpec((BLK, D), lambda i: (i, 0)),
        pl.BlockSpec((D, H), lambda i: (0, 0)),
    ],
    out_specs=pl.BlockSpec((BLK, H), lambda i: (i, 0)),
    out_shape=jax.ShapeDtypeStruct((N, H), jnp.float32),
)


def _mlp1_body(y_ref, p0_ref, p1_ref, b1_ref, w2_ref, b2_ref, o_ref):
    z = jnp.maximum(y_ref[...] + p0_ref[0] + p1_ref[0] + b1_ref[...], 0.0)
    t = jnp.dot(z, w2_ref[...], preferred_element_type=jnp.float32,
                         precision=lax.Precision.HIGHEST) + b2_ref[...]
    o_ref[...] = jnp.maximum(t, 0.0)


_mlp1 = pl.pallas_call(
    _mlp1_body,
    grid=(NBLK,),
    in_specs=[
        pl.BlockSpec((BLK, H), lambda i: (i, 0)),
        pl.BlockSpec((1, BLK, H), lambda i: (0, i, 0)),
        pl.BlockSpec((1, BLK, H), lambda i: (1, i, 0)),
        pl.BlockSpec((1, H), lambda i: (0, 0)),
        pl.BlockSpec((H, H), lambda i: (0, 0)),
        pl.BlockSpec((1, H), lambda i: (0, 0)),
    ],
    out_specs=pl.BlockSpec((BLK, H), lambda i: (i, 0)),
    out_shape=jax.ShapeDtypeStruct((N, H), jnp.float32),
)


def _mlp2_body(h_ref, q0_ref, q1_ref, b_ref, w1_ref, b1_ref, w2_ref, b2_ref,
               wl_ref, bl_ref, o_ref, sums, cnt):
    i = pl.program_id(0)

    @pl.when(i == 0)
    def _():
        sums[...] = jnp.zeros_like(sums)
        cnt[...] = jnp.zeros_like(cnt)

    z = h_ref[...] + q0_ref[0] + q1_ref[0]
    t = jnp.maximum(
        jnp.dot(z, w1_ref[...], preferred_element_type=jnp.float32,
                         precision=lax.Precision.HIGHEST)
        + b1_ref[...], 0.0)
    u = jnp.dot(t, w2_ref[...], preferred_element_type=jnp.float32,
                         precision=lax.Precision.HIGHEST) + b2_ref[...]

    gids = lax.broadcasted_iota(jnp.int32, (G, BLK), 0)
    onehot_t = (b_ref[0] == gids).astype(jnp.float32)
    sums[...] += jnp.dot(onehot_t, u, preferred_element_type=jnp.float32,
                         precision=lax.Precision.HIGHEST)
    cnt[...] += jnp.dot(onehot_t, jnp.ones((BLK, 1), jnp.float32),
                        preferred_element_type=jnp.float32,
                         precision=lax.Precision.HIGHEST)

    @pl.when(i == pl.num_programs(0) - 1)
    def _():
        v = jnp.dot(sums[...], wl_ref[...], preferred_element_type=jnp.float32,
                         precision=lax.Precision.HIGHEST)
        o_ref[...] = v / jnp.maximum(cnt[...], 1.0) + bl_ref[...]


_mlp2pool = pl.pallas_call(
    _mlp2_body,
    grid=(NBLK,),
    in_specs=[
        pl.BlockSpec((BLK, H), lambda i: (i, 0)),
        pl.BlockSpec((1, BLK, H), lambda i: (0, i, 0)),
        pl.BlockSpec((1, BLK, H), lambda i: (1, i, 0)),
        pl.BlockSpec((1, 1, BLK), lambda i: (i, 0, 0)),
        pl.BlockSpec((H, H), lambda i: (0, 0)),
        pl.BlockSpec((1, H), lambda i: (0, 0)),
        pl.BlockSpec((H, H), lambda i: (0, 0)),
        pl.BlockSpec((1, H), lambda i: (0, 0)),
        pl.BlockSpec((H, 1), lambda i: (0, 0)),
        pl.BlockSpec((1, 1), lambda i: (0, 0)),
    ],
    out_specs=pl.BlockSpec((G, 1), lambda i: (0, 0)),
    out_shape=jax.ShapeDtypeStruct((G, 1), jnp.float32),
    scratch_shapes=[
        pltpu.VMEM((G, H), jnp.float32),
        pltpu.VMEM((G, 1), jnp.float32),
    ],
)


def kernel(x, edge_index, batch, W1a, b1a, W2a, b2a, W1b, b1b, W2b, b2b, Wl, bl):
    src = edge_index[0].astype(jnp.int32)
    dst = edge_index[1].astype(jnp.int32)
    src3 = jnp.concatenate(
        [src, jnp.zeros((EPAD - E,), jnp.int32)]).reshape(NW, NCHUNK, K)
    dst3 = jnp.concatenate(
        [dst, jnp.full((EPAD - E,), N, jnp.int32)]).reshape(NW, NCHUNK, K)
    batch3 = batch.astype(jnp.int32).reshape(NBLK, 1, BLK)
    b1a2 = b1a.reshape(1, H)
    b2a2 = b2a.reshape(1, H)
    b1b2 = b1b.reshape(1, H)
    b2b2 = b2b.reshape(1, H)
    bl2 = bl.reshape(1, 1)

    y = _proj(x, W1a)
    p = _edge_agg(src3, dst3, y)
    h = _mlp1(y, p, p, b1a2, W2a, b2a2)
    q = _edge_agg(src3, dst3, h)
    out = _mlp2pool(h, q, q, batch3,
                    W1b, b1b2, W2b, b2b2, Wl, bl2)
    return out


# speedup vs baseline: 19.8432x; 1.0250x over previous
import functools

import jax
import jax.numpy as jnp
from jax import lax
from jax.experimental import pallas as pl
from jax.experimental.pallas import tpu as pltpu
from jax.experimental.pallas import tpu_sc as plsc

N = 10000
E = 320000
D = 128
H = 16
G = 64

NC = 2
NS = 16
NW = NC * NS
EPW = E // NW
K = 128
NCHUNK = 80
EPAD = NW * NCHUNK * K
NBUF = 8
NPAD = 10240
NPS = NPAD // NS

BLK = N
NBLK = N // BLK
MBLK = 2000
MGRID = N // MBLK


_SC_MESH = plsc.VectorSubcoreMesh(core_axis_name="c", subcore_axis_name="s")


@functools.partial(
    pl.kernel,
    out_type=jax.ShapeDtypeStruct((NC, NPAD, H), jnp.float32),
    mesh=_SC_MESH,
    scratch_types=[
        pltpu.VMEM((NCHUNK, K), jnp.int32),
        pltpu.VMEM((NCHUNK, K), jnp.int32),
        [pltpu.VMEM((K, H), jnp.float32) for _ in range(NBUF)],
        pltpu.VMEM_SHARED((NPAD, H), jnp.float32),
        pltpu.VMEM_SHARED((N, H), jnp.float32),
        [pltpu.SemaphoreType.DMA for _ in range(NBUF)],
        [pltpu.SemaphoreType.DMA for _ in range(NBUF)],
    ],
    compiler_params=pltpu.CompilerParams(use_tc_tiling_on_sc=False),
)
def _edge_agg(src_hbm, dst_hbm, y_hbm, z_hbm, out_hbm,
              src_all, dst_all, rows, acc_sh, y_sh, gsems, ssems):
    c = lax.axis_index("c")
    s = lax.axis_index("s")
    wid = c * NS + s

    YPS = N // NS
    stages = [
        pltpu.async_copy(src_hbm.at[wid], src_all, gsems[0]),
        pltpu.async_copy(dst_hbm.at[wid], dst_all, gsems[1]),
        pltpu.async_copy(y_hbm.at[pl.ds(s * YPS, YPS)],
                         y_sh.at[pl.ds(s * YPS, YPS)], gsems[2]),
        pltpu.async_copy(z_hbm, acc_sh.at[pl.ds(s * NPS, NPS)], gsems[3]),
    ]
    for d in stages:
        d.wait()
    plsc.subcore_barrier()

    def _iter(i, carry):
        j0 = i * NBUF
        gds = []
        for b in range(NBUF):
            @pl.when(i > 0)
            def _(b=b, j0=j0):
                pltpu.make_async_copy(
                    rows[b], acc_sh.at[dst_all.at[j0 - NBUF + b]], ssems[b]
                ).wait()

            gds.append(
                pltpu.async_copy(y_sh.at[src_all.at[j0 + b]], rows[b], gsems[b]))
        for b in range(NBUF):
            gds[b].wait()
            pltpu.async_copy(rows[b], acc_sh.at[dst_all.at[j0 + b]], ssems[b],
                             add=True)
        return carry

    lax.fori_loop(0, NCHUNK // NBUF, _iter, 0)
    for b in range(NBUF):
        pltpu.make_async_copy(
            rows[b], acc_sh.at[dst_all.at[NCHUNK - NBUF + b]], ssems[b]).wait()

    plsc.subcore_barrier()
    pltpu.sync_copy(acc_sh.at[pl.ds(s * NPS, NPS)],
                    out_hbm.at[c, pl.ds(s * NPS, NPS)])


def _proj_body(x_ref, w_ref, o_ref):
    o_ref[...] = jnp.dot(x_ref[...], w_ref[...],
                         preferred_element_type=jnp.float32,
                         precision=lax.Precision.HIGHEST)


_proj = pl.pallas_call(
    _proj_body,
    grid=(NBLK,),
    in_specs=[
        pl.BlockSpec((BLK, D), lambda i: (i, 0)),
        pl.BlockSpec((D, H), lambda i: (0, 0)),
    ],
    out_specs=pl.BlockSpec((BLK, H), lambda i: (i, 0)),
    out_shape=jax.ShapeDtypeStruct((N, H), jnp.float32),
)


def _mlp1_body(y_ref, p0_ref, p1_ref, b1_ref, w2_ref, b2_ref, o_ref):
    z = jnp.maximum(y_ref[...] + p0_ref[0] + p1_ref[0] + b1_ref[...], 0.0)
    t = jnp.dot(z, w2_ref[...], preferred_element_type=jnp.float32,
                         precision=lax.Precision.HIGHEST) + b2_ref[...]
    o_ref[...] = jnp.maximum(t, 0.0)


_mlp1 = pl.pallas_call(
    _mlp1_body,
    grid=(NBLK,),
    in_specs=[
        pl.BlockSpec((BLK, H), lambda i: (i, 0)),
        pl.BlockSpec((1, BLK, H), lambda i: (0, i, 0)),
        pl.BlockSpec((1, BLK, H), lambda i: (1, i, 0)),
        pl.BlockSpec((1, H), lambda i: (0, 0)),
        pl.BlockSpec((H, H), lambda i: (0, 0)),
        pl.BlockSpec((1, H), lambda i: (0, 0)),
    ],
    out_specs=pl.BlockSpec((BLK, H), lambda i: (i, 0)),
    out_shape=jax.ShapeDtypeStruct((N, H), jnp.float32),
)


def _mlp2_body(h_ref, q0_ref, q1_ref, b_ref, w1_ref, b1_ref, w2_ref, b2_ref,
               wl_ref, bl_ref, o_ref, sums, cnt):
    i = pl.program_id(0)

    @pl.when(i == 0)
    def _():
        sums[...] = jnp.zeros_like(sums)
        cnt[...] = jnp.zeros_like(cnt)

    z = h_ref[...] + q0_ref[0] + q1_ref[0]
    t = jnp.maximum(
        jnp.dot(z, w1_ref[...], preferred_element_type=jnp.float32,
                precision=lax.Precision.HIGHEST)
        + b1_ref[...], 0.0)
    u = jnp.dot(t, w2_ref[...], preferred_element_type=jnp.float32,
                precision=lax.Precision.HIGHEST) + b2_ref[...]

    gids = lax.broadcasted_iota(jnp.int32, (G, MBLK), 0)
    onehot_t = (b_ref[0] == gids).astype(jnp.float32)
    sums[...] += jnp.dot(onehot_t, u, preferred_element_type=jnp.float32,
                         precision=lax.Precision.HIGHEST)
    cnt[...] += jnp.dot(onehot_t, jnp.ones((MBLK, 1), jnp.float32),
                        preferred_element_type=jnp.float32,
                        precision=lax.Precision.HIGHEST)

    @pl.when(i == pl.num_programs(0) - 1)
    def _():
        v = jnp.dot(sums[...], wl_ref[...], preferred_element_type=jnp.float32,
                    precision=lax.Precision.HIGHEST)
        o_ref[...] = v / jnp.maximum(cnt[...], 1.0) + bl_ref[...]


_mlp2pool = pl.pallas_call(
    _mlp2_body,
    grid=(MGRID,),
    in_specs=[
        pl.BlockSpec((MBLK, H), lambda i: (i, 0)),
        pl.BlockSpec((1, MBLK, H), lambda i: (0, i, 0)),
        pl.BlockSpec((1, MBLK, H), lambda i: (1, i, 0)),
        pl.BlockSpec((1, 1, MBLK), lambda i: (i, 0, 0)),
        pl.BlockSpec((H, H), lambda i: (0, 0)),
        pl.BlockSpec((1, H), lambda i: (0, 0)),
        pl.BlockSpec((H, H), lambda i: (0, 0)),
        pl.BlockSpec((1, H), lambda i: (0, 0)),
        pl.BlockSpec((H, 1), lambda i: (0, 0)),
        pl.BlockSpec((1, 1), lambda i: (0, 0)),
    ],
    out_specs=pl.BlockSpec((G, 1), lambda i: (0, 0)),
    out_shape=jax.ShapeDtypeStruct((G, 1), jnp.float32),
    scratch_shapes=[
        pltpu.VMEM((G, H), jnp.float32),
        pltpu.VMEM((G, 1), jnp.float32),
    ],
)


def kernel(x, edge_index, batch, W1a, b1a, W2a, b2a, W1b, b1b, W2b, b2b, Wl, bl):
    src = edge_index[0].astype(jnp.int32)
    dst = edge_index[1].astype(jnp.int32)
    src3 = jnp.concatenate(
        [src, jnp.zeros((EPAD - E,), jnp.int32)]).reshape(NW, NCHUNK, K)
    dst3 = jnp.concatenate(
        [dst, jnp.full((EPAD - E,), N, jnp.int32)]).reshape(NW, NCHUNK, K)
    batch3 = batch.astype(jnp.int32).reshape(MGRID, 1, MBLK)
    b1a2 = b1a.reshape(1, H)
    b2a2 = b2a.reshape(1, H)
    b1b2 = b1b.reshape(1, H)
    b2b2 = b2b.reshape(1, H)
    bl2 = bl.reshape(1, 1)

    zfill = jnp.zeros((NPS, H), jnp.float32)
    y = _proj(x, W1a)
    p = _edge_agg(src3, dst3, y, zfill)
    h = _mlp1(y, p, p, b1a2, W2a, b2a2)
    q = _edge_agg(src3, dst3, h, zfill)
    out = _mlp2pool(h, q, q, batch3,
                    W1b, b1b2, W2b, b2b2, Wl, bl2)
    return out


# speedup vs baseline: 26.6106x; 1.3410x over previous
import functools

import jax
import jax.numpy as jnp
from jax import lax
from jax.experimental import pallas as pl
from jax.experimental.pallas import tpu as pltpu
from jax.experimental.pallas import tpu_sc as plsc

N = 10000
E = 320000
D = 128
H = 16
G = 64

NC = 2
NS = 16
NW = NC * NS
K = 128
NCHUNK = 80
EPAD = NW * NCHUNK * K
NBUF = 8
NPAD = 10240
NPS = NPAD // NS

P = 128 // H
NR = N // P
NRP = NPAD // P


_SC_MESH = plsc.VectorSubcoreMesh(core_axis_name="c", subcore_axis_name="s")


@functools.partial(
    pl.kernel,
    out_type=jax.ShapeDtypeStruct((NC, NPAD, H), jnp.float32),
    mesh=_SC_MESH,
    scratch_types=[
        pltpu.VMEM((NCHUNK, K), jnp.int32),
        pltpu.VMEM((NCHUNK, K), jnp.int32),
        [pltpu.VMEM((K, H), jnp.float32) for _ in range(NBUF)],
        pltpu.VMEM_SHARED((NPAD, H), jnp.float32),
        pltpu.VMEM_SHARED((N, H), jnp.float32),
        [pltpu.SemaphoreType.DMA for _ in range(NBUF)],
        [pltpu.SemaphoreType.DMA for _ in range(NBUF)],
    ],
    compiler_params=pltpu.CompilerParams(use_tc_tiling_on_sc=False),
)
def _edge_agg(src_hbm, dst_hbm, y_hbm, z_hbm, out_hbm,
              src_all, dst_all, rows, acc_sh, y_sh, gsems, ssems):
    c = lax.axis_index("c")
    s = lax.axis_index("s")
    wid = c * NS + s

    YPS = N // NS
    stages = [
        pltpu.async_copy(src_hbm.at[wid], src_all, gsems[0]),
        pltpu.async_copy(dst_hbm.at[wid], dst_all, gsems[1]),
        pltpu.async_copy(y_hbm.at[pl.ds(s * YPS, YPS)],
                         y_sh.at[pl.ds(s * YPS, YPS)], gsems[2]),
        pltpu.async_copy(z_hbm, acc_sh.at[pl.ds(s * NPS, NPS)], gsems[3]),
    ]
    for d in stages:
        d.wait()
    plsc.subcore_barrier()

    def _iter(i, carry):
        j0 = i * NBUF
        gds = []
        for b in range(NBUF):
            @pl.when(i > 0)
            def _(b=b, j0=j0):
                pltpu.make_async_copy(
                    rows[b], acc_sh.at[dst_all.at[j0 - NBUF + b]], ssems[b]
                ).wait()

            gds.append(
                pltpu.async_copy(y_sh.at[src_all.at[j0 + b]], rows[b], gsems[b]))
        for b in range(NBUF):
            gds[b].wait()
            pltpu.async_copy(rows[b], acc_sh.at[dst_all.at[j0 + b]], ssems[b],
                             add=True)
        return carry

    lax.fori_loop(0, NCHUNK // NBUF, _iter, 0)
    for b in range(NBUF):
        pltpu.make_async_copy(
            rows[b], acc_sh.at[dst_all.at[NCHUNK - NBUF + b]], ssems[b]).wait()

    plsc.subcore_barrier()
    pltpu.sync_copy(acc_sh.at[pl.ds(s * NPS, NPS)],
                    out_hbm.at[c, pl.ds(s * NPS, NPS)])


def _proj_body(x_ref, w_ref, o_ref):
    w = w_ref[...]
    parts = [
        jnp.dot(x_ref[:, a, :], w, preferred_element_type=jnp.float32,
                precision=lax.Precision.HIGHEST)
        for a in range(P)
    ]
    o_ref[...] = jnp.concatenate(parts, axis=1)


_proj = pl.pallas_call(
    _proj_body,
    grid=(1,),
    in_specs=[
        pl.BlockSpec((NR, P, D), lambda i: (0, 0, 0)),
        pl.BlockSpec((D, H), lambda i: (0, 0)),
    ],
    out_specs=pl.BlockSpec((NR, 128), lambda i: (0, 0)),
    out_shape=jax.ShapeDtypeStruct((NR, 128), jnp.float32),
)


def _mlp1_body(y_ref, p0_ref, p1_ref, b1_ref, w2_ref, b2_ref, o_ref):
    z = jnp.maximum(
        y_ref[...] + p0_ref[0, :NR, :] + p1_ref[0, :NR, :] + b1_ref[...], 0.0)
    t = jnp.dot(z, w2_ref[...], preferred_element_type=jnp.float32,
                precision=lax.Precision.HIGHEST) + b2_ref[...]
    o_ref[...] = jnp.maximum(t, 0.0)


_mlp1 = pl.pallas_call(
    _mlp1_body,
    grid=(1,),
    in_specs=[
        pl.BlockSpec((NR, 128), lambda i: (0, 0)),
        pl.BlockSpec((1, NRP, 128), lambda i: (0, 0, 0)),
        pl.BlockSpec((1, NRP, 128), lambda i: (1, 0, 0)),
        pl.BlockSpec((1, 128), lambda i: (0, 0)),
        pl.BlockSpec((128, 128), lambda i: (0, 0)),
        pl.BlockSpec((1, 128), lambda i: (0, 0)),
    ],
    out_specs=pl.BlockSpec((NR, 128), lambda i: (0, 0)),
    out_shape=jax.ShapeDtypeStruct((NR, 128), jnp.float32),
)


def _mlp2_body(h_ref, q0_ref, q1_ref, b_ref, w1_ref, b1_ref, w2_ref, b2_ref,
               wl_ref, bl_ref, o_ref):
    z = h_ref[...] + q0_ref[0, :NR, :] + q1_ref[0, :NR, :]
    t = jnp.maximum(
        jnp.dot(z, w1_ref[...], preferred_element_type=jnp.float32,
                precision=lax.Precision.HIGHEST)
        + b1_ref[...], 0.0)
    u = jnp.dot(t, w2_ref[...], preferred_element_type=jnp.float32,
                precision=lax.Precision.HIGHEST) + b2_ref[...]

    gid = lax.broadcasted_iota(jnp.int32, (G, NR), 0)
    sums = jnp.zeros((G, H), jnp.float32)
    cnt = jnp.zeros((G, 1), jnp.float32)
    ones = jnp.ones((NR, 1), jnp.float32)
    for a in range(P):
        onehot_t = (b_ref[a:a + 1, :] == gid).astype(jnp.float32)
        sums += jnp.dot(onehot_t, u[:, a * H:(a + 1) * H],
                        preferred_element_type=jnp.float32,
                        precision=lax.Precision.HIGHEST)
        cnt += jnp.dot(onehot_t, ones, preferred_element_type=jnp.float32,
                       precision=lax.Precision.HIGHEST)

    v = jnp.dot(sums, wl_ref[...], preferred_element_type=jnp.float32,
                precision=lax.Precision.HIGHEST)
    o_ref[...] = v / jnp.maximum(cnt, 1.0) + bl_ref[...]


_mlp2pool = pl.pallas_call(
    _mlp2_body,
    grid=(1,),
    in_specs=[
        pl.BlockSpec((NR, 128), lambda i: (0, 0)),
        pl.BlockSpec((1, NRP, 128), lambda i: (0, 0, 0)),
        pl.BlockSpec((1, NRP, 128), lambda i: (1, 0, 0)),
        pl.BlockSpec((P, NR), lambda i: (0, 0)),
        pl.BlockSpec((128, 128), lambda i: (0, 0)),
        pl.BlockSpec((1, 128), lambda i: (0, 0)),
        pl.BlockSpec((128, 128), lambda i: (0, 0)),
        pl.BlockSpec((1, 128), lambda i: (0, 0)),
        pl.BlockSpec((H, 1), lambda i: (0, 0)),
        pl.BlockSpec((1, 1), lambda i: (0, 0)),
    ],
    out_specs=pl.BlockSpec((G, 1), lambda i: (0, 0)),
    out_shape=jax.ShapeDtypeStruct((G, 1), jnp.float32),
)


def _blockdiag(w):
    return jnp.kron(jnp.eye(P, dtype=jnp.float32), w)


def kernel(x, edge_index, batch, W1a, b1a, W2a, b2a, W1b, b1b, W2b, b2b, Wl, bl):
    src = edge_index[0].astype(jnp.int32)
    dst = edge_index[1].astype(jnp.int32)
    src3 = jnp.concatenate(
        [src, jnp.zeros((EPAD - E,), jnp.int32)]).reshape(NW, NCHUNK, K)
    dst3 = jnp.concatenate(
        [dst, jnp.full((EPAD - E,), N, jnp.int32)]).reshape(NW, NCHUNK, K)
    batch_t = batch.astype(jnp.int32).reshape(NR, P).T
    x3 = x.reshape(NR, P, D)

    w2a_bd = _blockdiag(W2a)
    w1b_bd = _blockdiag(W1b)
    w2b_bd = _blockdiag(W2b)
    b1a_t = jnp.tile(b1a, P).reshape(1, 128)
    b2a_t = jnp.tile(b2a, P).reshape(1, 128)
    b1b_t = jnp.tile(b1b, P).reshape(1, 128)
    b2b_t = jnp.tile(b2b, P).reshape(1, 128)
    bl2 = bl.reshape(1, 1)
    zfill = jnp.zeros((NPS, H), jnp.float32)

    y = _proj(x3, W1a)
    p = _edge_agg(src3, dst3, y.reshape(N, H), zfill)
    p_pk = p.reshape(NC, NRP, 128)
    h = _mlp1(y, p_pk, p_pk, b1a_t, w2a_bd, b2a_t)
    q = _edge_agg(src3, dst3, h.reshape(N, H), zfill)
    q_pk = q.reshape(NC, NRP, 128)
    out = _mlp2pool(h, q_pk, q_pk, batch_t,
                    w1b_bd, b1b_t, w2b_bd, b2b_t, Wl, bl2)
    return out


# speedup vs baseline: 26.6986x; 1.0033x over previous
import functools

import jax
import jax.numpy as jnp
from jax import lax
from jax.experimental import pallas as pl
from jax.experimental.pallas import tpu as pltpu
from jax.experimental.pallas import tpu_sc as plsc

N = 10000
E = 320000
D = 128
H = 16
G = 64

NC = 2
NS = 16
NW = NC * NS
K = 128
NCHUNK = 80
EPAD = NW * NCHUNK * K
NBUF = 8
NPAD = 10240
NPS = NPAD // NS

P = 128 // H
NR = N // P
NRP = NPAD // P


_SC_MESH = plsc.VectorSubcoreMesh(core_axis_name="c", subcore_axis_name="s")


@functools.partial(
    pl.kernel,
    out_type=jax.ShapeDtypeStruct((NC, NPAD, H), jnp.float32),
    mesh=_SC_MESH,
    scratch_types=[
        pltpu.VMEM((NCHUNK, K), jnp.int32),
        pltpu.VMEM((NCHUNK, K), jnp.int32),
        [pltpu.VMEM((K, H), jnp.float32) for _ in range(NBUF)],
        pltpu.VMEM_SHARED((NPAD, H), jnp.float32),
        pltpu.VMEM_SHARED((N, H), jnp.float32),
        [pltpu.SemaphoreType.DMA for _ in range(NBUF)],
        [pltpu.SemaphoreType.DMA for _ in range(NBUF)],
    ],
    compiler_params=pltpu.CompilerParams(use_tc_tiling_on_sc=False),
)
def _edge_agg(src_hbm, dst_hbm, y_hbm, z_hbm, out_hbm,
              src_all, dst_all, rows, acc_sh, y_sh, gsems, ssems):
    c = lax.axis_index("c")
    s = lax.axis_index("s")
    wid = c * NS + s

    YPS = N // NS
    stages = [
        pltpu.async_copy(src_hbm.at[wid], src_all, gsems[0]),
        pltpu.async_copy(dst_hbm.at[wid], dst_all, gsems[1]),
        pltpu.async_copy(y_hbm.at[pl.ds(s * YPS, YPS)],
                         y_sh.at[pl.ds(s * YPS, YPS)], gsems[2]),
        pltpu.async_copy(z_hbm, acc_sh.at[pl.ds(s * NPS, NPS)], gsems[3]),
    ]
    for d in stages:
        d.wait()
    plsc.subcore_barrier()

    def _iter(i, carry):
        j0 = i * NBUF
        gds = []
        for b in range(NBUF):
            @pl.when(i > 0)
            def _(b=b, j0=j0):
                pltpu.make_async_copy(
                    rows[b], acc_sh.at[dst_all.at[j0 - NBUF + b]], ssems[b]
                ).wait()

            gds.append(
                pltpu.async_copy(y_sh.at[src_all.at[j0 + b]], rows[b], gsems[b]))
        for b in range(NBUF):
            gds[b].wait()
            pltpu.async_copy(rows[b], acc_sh.at[dst_all.at[j0 + b]], ssems[b],
                             add=True)
        return carry

    lax.fori_loop(0, NCHUNK // NBUF, _iter, 0)
    for b in range(NBUF):
        pltpu.make_async_copy(
            rows[b], acc_sh.at[dst_all.at[NCHUNK - NBUF + b]], ssems[b]).wait()

    plsc.subcore_barrier()
    pltpu.sync_copy(acc_sh.at[pl.ds(s * NPS, NPS)],
                    out_hbm.at[c, pl.ds(s * NPS, NPS)])


def _proj_body(x_ref, w_ref, o_ref):
    w = w_ref[...]
    x3 = x_ref[...].reshape(NR, P, D)
    parts = [
        jnp.dot(x3[:, a, :], w, preferred_element_type=jnp.float32,
                precision=lax.Precision.HIGHEST)
        for a in range(P)
    ]
    o_ref[...] = jnp.concatenate(parts, axis=1)


_proj = pl.pallas_call(
    _proj_body,
    grid=(1,),
    in_specs=[
        pl.BlockSpec((N, D), lambda i: (0, 0)),
        pl.BlockSpec((D, H), lambda i: (0, 0)),
    ],
    out_specs=pl.BlockSpec((NR, 128), lambda i: (0, 0)),
    out_shape=jax.ShapeDtypeStruct((NR, 128), jnp.float32),
)


def _mlp1_body(y_ref, p0_ref, p1_ref, b1_ref, w2_ref, b2_ref, o_ref):
    z = jnp.maximum(
        y_ref[...] + p0_ref[0, :NR, :] + p1_ref[0, :NR, :] + b1_ref[...], 0.0)
    t = jnp.dot(z, w2_ref[...], preferred_element_type=jnp.float32,
                precision=lax.Precision.HIGHEST) + b2_ref[...]
    o_ref[...] = jnp.maximum(t, 0.0)


_mlp1 = pl.pallas_call(
    _mlp1_body,
    grid=(1,),
    in_specs=[
        pl.BlockSpec((NR, 128), lambda i: (0, 0)),
        pl.BlockSpec((1, NRP, 128), lambda i: (0, 0, 0)),
        pl.BlockSpec((1, NRP, 128), lambda i: (1, 0, 0)),
        pl.BlockSpec((1, 128), lambda i: (0, 0)),
        pl.BlockSpec((128, 128), lambda i: (0, 0)),
        pl.BlockSpec((1, 128), lambda i: (0, 0)),
    ],
    out_specs=pl.BlockSpec((NR, 128), lambda i: (0, 0)),
    out_shape=jax.ShapeDtypeStruct((NR, 128), jnp.float32),
)


def _mlp2_body(h_ref, q0_ref, q1_ref, b_ref, w1_ref, b1_ref, w2_ref, b2_ref,
               wl_ref, bl_ref, o_ref):
    z = h_ref[...] + q0_ref[0, :NR, :] + q1_ref[0, :NR, :]
    t = jnp.maximum(
        jnp.dot(z, w1_ref[...], preferred_element_type=jnp.float32,
                precision=lax.Precision.HIGHEST)
        + b1_ref[...], 0.0)
    u = jnp.dot(t, w2_ref[...], preferred_element_type=jnp.float32,
                precision=lax.Precision.HIGHEST) + b2_ref[...]

    gid = lax.broadcasted_iota(jnp.int32, (G, NR), 0)
    sums = jnp.zeros((G, H), jnp.float32)
    cnt = jnp.zeros((G, 1), jnp.float32)
    ones = jnp.ones((NR, 1), jnp.float32)
    for a in range(P):
        onehot_t = (b_ref[a:a + 1, :] == gid).astype(jnp.float32)
        sums += jnp.dot(onehot_t, u[:, a * H:(a + 1) * H],
                        preferred_element_type=jnp.float32,
                        precision=lax.Precision.HIGHEST)
        cnt += jnp.dot(onehot_t, ones, preferred_element_type=jnp.float32,
                       precision=lax.Precision.HIGHEST)

    v = jnp.dot(sums, wl_ref[...], preferred_element_type=jnp.float32,
                precision=lax.Precision.HIGHEST)
    o_ref[...] = v / jnp.maximum(cnt, 1.0) + bl_ref[...]


_mlp2pool = pl.pallas_call(
    _mlp2_body,
    grid=(1,),
    in_specs=[
        pl.BlockSpec((NR, 128), lambda i: (0, 0)),
        pl.BlockSpec((1, NRP, 128), lambda i: (0, 0, 0)),
        pl.BlockSpec((1, NRP, 128), lambda i: (1, 0, 0)),
        pl.BlockSpec((P, NR), lambda i: (0, 0)),
        pl.BlockSpec((128, 128), lambda i: (0, 0)),
        pl.BlockSpec((1, 128), lambda i: (0, 0)),
        pl.BlockSpec((128, 128), lambda i: (0, 0)),
        pl.BlockSpec((1, 128), lambda i: (0, 0)),
        pl.BlockSpec((H, 1), lambda i: (0, 0)),
        pl.BlockSpec((1, 1), lambda i: (0, 0)),
    ],
    out_specs=pl.BlockSpec((G, 1), lambda i: (0, 0)),
    out_shape=jax.ShapeDtypeStruct((G, 1), jnp.float32),
)


def _blockdiag(w):
    return jnp.kron(jnp.eye(P, dtype=jnp.float32), w)


def kernel(x, edge_index, batch, W1a, b1a, W2a, b2a, W1b, b1b, W2b, b2b, Wl, bl):
    src = edge_index[0].astype(jnp.int32)
    dst = edge_index[1].astype(jnp.int32)
    src3 = jnp.concatenate(
        [src, jnp.zeros((EPAD - E,), jnp.int32)]).reshape(NW, NCHUNK, K)
    dst3 = jnp.concatenate(
        [dst, jnp.full((EPAD - E,), N, jnp.int32)]).reshape(NW, NCHUNK, K)
    batch_t = batch.astype(jnp.int32).reshape(NR, P).T

    w2a_bd = _blockdiag(W2a)
    w1b_bd = _blockdiag(W1b)
    w2b_bd = _blockdiag(W2b)
    b1a_t = jnp.tile(b1a, P).reshape(1, 128)
    b2a_t = jnp.tile(b2a, P).reshape(1, 128)
    b1b_t = jnp.tile(b1b, P).reshape(1, 128)
    b2b_t = jnp.tile(b2b, P).reshape(1, 128)
    bl2 = bl.reshape(1, 1)
    zfill = jnp.zeros((NPS, H), jnp.float32)

    y = _proj(x, W1a)
    p = _edge_agg(src3, dst3, y.reshape(N, H), zfill)
    p_pk = p.reshape(NC, NRP, 128)
    h = _mlp1(y, p_pk, p_pk, b1a_t, w2a_bd, b2a_t)
    q = _edge_agg(src3, dst3, h.reshape(N, H), zfill)
    q_pk = q.reshape(NC, NRP, 128)
    out = _mlp2pool(h, q_pk, q_pk, batch_t,
                    w1b_bd, b1b_t, w2b_bd, b2b_t, Wl, bl2)
    return out
